# baseline (speedup 1.0000x reference)
import functools

import jax
import jax.numpy as jnp
from jax import lax
from jax.experimental import pallas as pl
from jax.experimental.pallas import tpu as pltpu
from jax.experimental.pallas import tpu_sc as plsc

NC = 2
NS = 16
NW = NC * NS
CHUNK = 128
BROWS = 80


def _sc_hops(e, n_nodes, n_pad, d, trip):
  nblocks = n_pad // BROWS
  wblocks = n_nodes // BROWS
  epad = trip * NW * CHUNK
  mesh = plsc.VectorSubcoreMesh(core_axis_name="c", subcore_axis_name="s")

  @functools.partial(
      pl.kernel,
      out_type=jax.ShapeDtypeStruct((NC, n_nodes, d), jnp.float32),
      mesh=mesh,
      scratch_types=[
          [pltpu.VMEM((CHUNK,), jnp.int32)] * 2,
          [pltpu.VMEM((CHUNK,), jnp.int32)] * 2,
          pltpu.VMEM((CHUNK, d), jnp.float32),
          pltpu.VMEM((CHUNK, d), jnp.float32),
          pltpu.VMEM((8, d), jnp.float32),
          pltpu.VMEM_SHARED((n_pad, d), jnp.float32),
          [pltpu.SemaphoreType.DMA] * 2,
          pltpu.SemaphoreType.DMA,
      ],
  )
  def sc_kernel(x0_hbm, x1_hbm, x2_hbm, ea_hbm, ei0_hbm, ei1_hbm, ei2_hbm,
                out_hbm, src_v, dst_v, msg_v, ea_v, zero_v, acc, isem, gsem):

    c = lax.axis_index("c")
    s = lax.axis_index("s")
    w = c * NS + s

    zvec = jnp.zeros((16,), jnp.float32)
    for k in range(d // 16):
      for r in range(8):
        zero_v[r, pl.ds(16 * k, 16)] = zvec

    def zero_body(j, carry):
      blk = s + j * NS
      for m in range(BROWS // 8):
        pltpu.sync_copy(zero_v, acc.at[pl.ds(blk * BROWS + m * 8, 8), :])
      return carry
    lax.fori_loop(0, (nblocks - s + NS - 1) // NS, zero_body, 0)

    plsc.subcore_barrier()

    def hop(x_hbm, ei_hbm, with_ea):
      assert trip >= 4 and trip % 2 == 0

      def fire_idx(i, b):
        base = (w + i * NW) * CHUNK
        pltpu.async_copy(ei_hbm.at[pl.ds(base, CHUNK)], src_v[b], isem[b])
        pltpu.async_copy(ei_hbm.at[pl.ds(epad + base, CHUNK)], dst_v[b], isem[b])

      def wait_idx(b):
        pltpu.make_async_copy(ei_hbm.at[pl.ds(0, CHUNK)], src_v[b], isem[b]).wait()
        pltpu.make_async_copy(ei_hbm.at[pl.ds(0, CHUNK)], dst_v[b], isem[b]).wait()

      def run_iter(i, b, first=False, last=False):
        if not first:
          wait_idx(b)
        if not last:
          fire_idx(i + 1, 1 - b)
        pltpu.async_copy(x_hbm.at[src_v[b]], msg_v, gsem).wait()
        if with_ea:
          base = jnp.minimum((w + i * NW) * CHUNK, e - CHUNK)
          pltpu.async_copy(ea_hbm.at[pl.ds(base, CHUNK), :], ea_v, gsem).wait()
        pltpu.sync_copy(msg_v, acc.at[dst_v[b]], add=True)
        if with_ea:
          pltpu.sync_copy(ea_v, acc.at[dst_v[b]], add=True)

      base0 = w * CHUNK
      pltpu.sync_copy(ei_hbm.at[pl.ds(base0, CHUNK)], src_v[0])
      pltpu.sync_copy(ei_hbm.at[pl.ds(epad + base0, CHUNK)], dst_v[0])

      run_iter(0, 0, first=True)

      def loop_body(j, carry):
        run_iter(2 * j + 1, 1)
        run_iter(2 * j + 2, 0)
        return carry
      lax.fori_loop(0, (trip - 2) // 2, loop_body, 0)

      run_iter(trip - 1, 1, last=True)

    hop(x0_hbm, ei0_hbm, True)
    hop(x1_hbm, ei1_hbm, False)
    hop(x2_hbm, ei2_hbm, False)

    plsc.subcore_barrier()

    def write_body(j, carry):
      blk = s + j * NS
      pltpu.sync_copy(acc.at[pl.ds(blk * BROWS, BROWS), :],
                      out_hbm.at[c, pl.ds(blk * BROWS, BROWS), :])
      return carry
    lax.fori_loop(0, (wblocks - s + NS - 1) // NS, write_body, 0)

  return sc_kernel


def _mlp_body(p_ref, x0_ref, w1_ref, b1_ref, g1_ref, be1_ref,
              w2_ref, b2_ref, g2_ref, be2_ref, o_ref):
  r = x0_ref[...] + p_ref[0] + p_ref[1]
  h = jnp.dot(r, w1_ref[...], preferred_element_type=jnp.float32) + b1_ref[...]
  mu = jnp.mean(h, axis=0, keepdims=True)
  var = jnp.mean(jnp.square(h - mu), axis=0, keepdims=True)
  h = jnp.maximum((h - mu) * lax.rsqrt(var + 1e-5) * g1_ref[...] + be1_ref[...], 0.0)
  h = jnp.dot(h, w2_ref[...], preferred_element_type=jnp.float32) + b2_ref[...]
  mu = jnp.mean(h, axis=0, keepdims=True)
  var = jnp.mean(jnp.square(h - mu), axis=0, keepdims=True)
  o_ref[...] = jnp.maximum((h - mu) * lax.rsqrt(var + 1e-5) * g2_ref[...] + be2_ref[...], 0.0)


def _pad_indices(ei, epad, n_dummy):
  e = ei.shape[1]
  pad = epad - e
  src = jnp.concatenate([ei[0], jnp.zeros((pad,), jnp.int32)])
  dst = jnp.concatenate([ei[1], jnp.full((pad,), n_dummy, jnp.int32)])
  return jnp.concatenate([src, dst])


def kernel(x0, x1, x2, edge_attr, W1, b1, g1, be1, W2, b2, g2, be2,
           edge_index0, edge_index1, edge_index2):
  n, d = x0.shape
  e = edge_index0.shape[1]
  assert n % BROWS == 0
  nchunks = -(-e // CHUNK)
  nchunks = -(-nchunks // (4 * NW)) * (4 * NW)
  trip = nchunks // NW
  epad = nchunks * CHUNK
  n_pad = n + 8

  ei0 = _pad_indices(edge_index0, epad, n)
  ei1 = _pad_indices(edge_index1, epad, n)
  ei2 = _pad_indices(edge_index2, epad, n)

  parts = _sc_hops(e, n, n_pad, d, trip)(
      x0, x1, x2, edge_attr, ei0, ei1, ei2)

  out = pl.pallas_call(
      _mlp_body,
      out_shape=jax.ShapeDtypeStruct((n, d), jnp.float32),
  )(parts, x0, W1.T, b1.reshape(1, d), g1.reshape(1, d), be1.reshape(1, d),
    W2.T, b2.reshape(1, d), g2.reshape(1, d), be2.reshape(1, d))
  return out


# speedup vs baseline: 1.6694x; 1.6694x over previous
import functools

import jax
import jax.numpy as jnp
from jax import lax
from jax.experimental import pallas as pl
from jax.experimental.pallas import tpu as pltpu
from jax.experimental.pallas import tpu_sc as plsc

NC = 2
NS = 16
CHUNK = 128


def _sc_hops(nchunks, n_nodes, d):
  brows = 80
  nblocks = n_nodes // brows
  assert n_nodes % brows == 0
  mesh = plsc.VectorSubcoreMesh(core_axis_name="c", subcore_axis_name="s")

  @functools.partial(
      pl.kernel,
      out_type=jax.ShapeDtypeStruct((NC, n_nodes, d), jnp.float32),
      mesh=mesh,
      scratch_types=[
          pltpu.VMEM((CHUNK,), jnp.int32),
          pltpu.VMEM((CHUNK,), jnp.int32),
          pltpu.VMEM((CHUNK, d), jnp.float32),
          pltpu.VMEM((CHUNK, d), jnp.float32),
          pltpu.VMEM((16, d), jnp.float32),
          pltpu.VMEM_SHARED((n_nodes, d), jnp.float32),
          pltpu.SemaphoreType.DMA,
      ],
  )
  def sc_kernel(x0_hbm, x1_hbm, x2_hbm, ea_hbm, ei0_hbm, ei1_hbm, ei2_hbm,
                out_hbm, src_v, dst_v, msg_v, ea_v, zero_v, acc, sem):
    c = lax.axis_index("c")
    s = lax.axis_index("s")
    w = c * NS + s

    zvec = jnp.zeros((16,), jnp.float32)
    for r in range(16):
      for k in range(d // 16):
        zero_v[r, pl.ds(16 * k, 16)] = zvec

    trip_b = (nblocks - s + NS - 1) // NS

    def zero_body(j, carry):
      blk = s + j * NS
      for m in range(brows // 16):
        pltpu.sync_copy(zero_v, acc.at[pl.ds(blk * brows + m * 16, 16), :])
      return carry
    lax.fori_loop(0, trip_b, zero_body, 0)

    plsc.subcore_barrier()

    trip = (nchunks - w + NC * NS - 1) // (NC * NS)

    def make_body(x_hbm, ei_hbm, with_ea):
      def body(i, carry):
        base = (w + i * (NC * NS)) * CHUNK
        pltpu.sync_copy(ei_hbm.at[pl.ds(base, CHUNK)], src_v)
        pltpu.sync_copy(ei_hbm.at[pl.ds(nchunks * CHUNK + base, CHUNK)], dst_v)
        pltpu.async_copy(x_hbm.at[src_v], msg_v, sem).wait()
        pltpu.sync_copy(msg_v, acc.at[dst_v], add=True)
        if with_ea:
          pltpu.sync_copy(ea_hbm.at[pl.ds(base, CHUNK), :], ea_v)
          pltpu.sync_copy(ea_v, acc.at[dst_v], add=True)
        return carry
      return body

    lax.fori_loop(0, trip, make_body(x0_hbm, ei0_hbm, True), 0)
    lax.fori_loop(0, trip, make_body(x1_hbm, ei1_hbm, False), 0)
    lax.fori_loop(0, trip, make_body(x2_hbm, ei2_hbm, False), 0)

    plsc.subcore_barrier()

    def write_body(j, carry):
      blk = s + j * NS
      pltpu.sync_copy(acc.at[pl.ds(blk * brows, brows), :],
                      out_hbm.at[c, pl.ds(blk * brows, brows), :])
      return carry
    lax.fori_loop(0, trip_b, write_body, 0)

  return sc_kernel


def _mlp_body(p_ref, x0_ref, w1_ref, b1_ref, g1_ref, be1_ref,
              w2_ref, b2_ref, g2_ref, be2_ref, o_ref):
  r = x0_ref[...] + p_ref[0] + p_ref[1]
  h = jnp.dot(r, w1_ref[...], preferred_element_type=jnp.float32) + b1_ref[...]
  mu = jnp.mean(h, axis=0, keepdims=True)
  var = jnp.mean(jnp.square(h - mu), axis=0, keepdims=True)
  h = jnp.maximum((h - mu) * lax.rsqrt(var + 1e-5) * g1_ref[...] + be1_ref[...], 0.0)
  h = jnp.dot(h, w2_ref[...], preferred_element_type=jnp.float32) + b2_ref[...]
  mu = jnp.mean(h, axis=0, keepdims=True)
  var = jnp.mean(jnp.square(h - mu), axis=0, keepdims=True)
  o_ref[...] = jnp.maximum((h - mu) * lax.rsqrt(var + 1e-5) * g2_ref[...] + be2_ref[...], 0.0)


def kernel(x0, x1, x2, edge_attr, W1, b1, g1, be1, W2, b2, g2, be2,
           edge_index0, edge_index1, edge_index2):
  n, d = x0.shape
  e = edge_index0.shape[1]
  assert e % CHUNK == 0 and n % 80 == 0

  parts = _sc_hops(e // CHUNK, n, d)(
      x0, x1, x2, edge_attr,
      edge_index0.reshape(-1), edge_index1.reshape(-1), edge_index2.reshape(-1))

  out = pl.pallas_call(
      _mlp_body,
      out_shape=jax.ShapeDtypeStruct((n, d), jnp.float32),
  )(parts, x0, W1.T, b1.reshape(1, d), g1.reshape(1, d), be1.reshape(1, d),
    W2.T, b2.reshape(1, d), g2.reshape(1, d), be2.reshape(1, d))
  return out
